# Initial kernel scaffold; baseline (speedup 1.0000x reference)
import functools

import jax
import jax.numpy as jnp
from jax import lax
from jax.experimental import pallas as pl
from jax.experimental.pallas import tpu as pltpu
from jax.experimental.pallas import tpu_sc as plsc

B = 16384
C = 1000
LANES = 16
NC, NS = 2, 16
NW = NC * NS
RPW = B // NW
RBLK = 32
NBLK = RPW // RBLK
_CHUNKS = [16 * j for j in range(C // 16)] + [C - LANES]

_mesh = plsc.VectorSubcoreMesh(core_axis_name="c", subcore_axis_name="s")


@functools.partial(
    pl.kernel,
    out_type=jax.ShapeDtypeStruct((B,), jnp.float32),
    mesh=_mesh,
    scratch_types=[
        pltpu.VMEM((RBLK, C), jnp.float32),
        pltpu.VMEM((RPW,), jnp.int32),
        pltpu.VMEM((RPW,), jnp.float32),
        pltpu.VMEM((LANES,), jnp.float32),
        pltpu.SemaphoreType.DMA,
    ],
)
def _cw_kernel(pred_hbm, y_hbm, out_hbm, buf, y_v, loss_v, tmp_v, sem):
    wid = lax.axis_index("s") * NC + lax.axis_index("c")
    base = pl.multiple_of(wid * RPW, RPW)
    pltpu.sync_copy(y_hbm.at[pl.ds(base, RPW)], y_v)

    neg_inf = jnp.full((LANES,), -jnp.inf, dtype=jnp.float32)

    def block_body(g, carry):
        row0 = pl.multiple_of(base + g * RBLK, RBLK)
        pltpu.async_copy(pred_hbm.at[pl.ds(row0, RBLK)], buf, sem).wait()
        for sub in range(RBLK // LANES):
            off = pl.multiple_of(g * RBLK + sub * LANES, LANES)
            rvec = lax.iota(jnp.int32, LANES) + (sub * LANES)
            yvec = y_v[pl.ds(off, LANES)]
            cls = plsc.load_gather(buf, [rvec, yvec])
            plsc.store_scatter(buf, [rvec, yvec], neg_inf)
            for r in range(LANES):
                row = sub * LANES + r
                acc = [
                    buf[row, pl.ds(_CHUNKS[0], LANES)],
                    buf[row, pl.ds(_CHUNKS[1], LANES)],
                    buf[row, pl.ds(_CHUNKS[2], LANES)],
                ]
                for j in range(3, len(_CHUNKS)):
                    k = j % 3
                    acc[k] = jnp.maximum(acc[k], buf[row, pl.ds(_CHUNKS[j], LANES)])
                m = jnp.maximum(acc[0], jnp.maximum(acc[1], acc[2]))
                tmp_v[r] = jnp.max(m)
            loss_v[pl.ds(off, LANES)] = tmp_v[...] - cls
        return carry

    lax.fori_loop(0, NBLK, block_body, 0)
    pltpu.sync_copy(loss_v, out_hbm.at[pl.ds(base, RPW)])


def kernel(pred, y):
    return _cw_kernel(pred, y.astype(jnp.int32))


# speedup vs baseline: 13.6613x; 13.6613x over previous
import functools

import jax
import jax.numpy as jnp
from jax import lax
from jax.experimental import pallas as pl
from jax.experimental.pallas import tpu as pltpu
from jax.experimental.pallas import tpu_sc as plsc

B = 16384
C = 1000
LANES = 16
NC, NS = 2, 16
NW = NC * NS
RPW = B // NW
RBLK = 32
NBLK = RPW // RBLK
_CHUNKS = [16 * j for j in range(C // 16)] + [C - LANES]

_mesh = plsc.VectorSubcoreMesh(core_axis_name="c", subcore_axis_name="s")


@functools.partial(
    pl.kernel,
    out_type=jax.ShapeDtypeStruct((B,), jnp.float32),
    mesh=_mesh,
    scratch_types=[
        pltpu.VMEM((RBLK, C), jnp.float32),
        pltpu.VMEM((RPW,), jnp.int32),
        pltpu.VMEM((RPW,), jnp.float32),
        pltpu.SemaphoreType.DMA,
    ],
    compiler_params=pltpu.CompilerParams(
        use_tc_tiling_on_sc=False, needs_layout_passes=False
    ),
)
def _cw_kernel(pred_hbm, y_hbm, out_hbm, buf, y_v, loss_v, sem):
    wid = lax.axis_index("s") * NC + lax.axis_index("c")
    base = pl.multiple_of(wid * RPW, RPW)
    pltpu.sync_copy(y_hbm.at[pl.ds(base, RPW)], y_v)

    neg_inf = jnp.full((LANES,), -jnp.inf, dtype=jnp.float32)

    def block_body(g, carry):
        row0 = pl.multiple_of(base + g * RBLK, RBLK)
        pltpu.async_copy(pred_hbm.at[pl.ds(row0, RBLK)], buf, sem).wait()
        for sub in range(RBLK // LANES):
            off = pl.multiple_of(g * RBLK + sub * LANES, LANES)
            rvec = lax.iota(jnp.int32, LANES) + (sub * LANES)
            yvec = y_v[pl.ds(off, LANES)]
            cls = plsc.load_gather(buf, [rvec, yvec])
            plsc.store_scatter(buf, [rvec, yvec], neg_inf)
            rowmax = neg_inf
            lane = lax.iota(jnp.int32, LANES)
            for r in range(LANES):
                row = sub * LANES + r
                acc = [
                    buf[row, pl.ds(_CHUNKS[0], LANES)],
                    buf[row, pl.ds(_CHUNKS[1], LANES)],
                    buf[row, pl.ds(_CHUNKS[2], LANES)],
                ]
                for j in range(3, len(_CHUNKS)):
                    k = j % 3
                    acc[k] = jnp.maximum(acc[k], buf[row, pl.ds(_CHUNKS[j], LANES)])
                m = jnp.maximum(acc[0], jnp.maximum(acc[1], acc[2]))
                rowmax = jnp.where(lane == r, jnp.max(m), rowmax)
            loss_v[pl.ds(off, LANES)] = rowmax - cls
        return carry

    lax.fori_loop(0, NBLK, block_body, 0)
    pltpu.sync_copy(loss_v, out_hbm.at[pl.ds(base, RPW)])


def kernel(pred, y):
    return _cw_kernel(pred, y.astype(jnp.int32))


# speedup vs baseline: 37.7330x; 2.7620x over previous
import functools

import jax
import jax.numpy as jnp
from jax import lax
from jax.experimental import pallas as pl
from jax.experimental.pallas import tpu as pltpu
from jax.experimental.pallas import tpu_sc as plsc

B = 16384
C = 1000
LANES = 16
NC, NS = 2, 16
NW = NC * NS
RPW = B // NW
RBLK = 32
NBLK = RPW // RBLK
NPAIR = C // 32
HALF = 8

_mesh = plsc.VectorSubcoreMesh(core_axis_name="c", subcore_axis_name="s")


@functools.partial(
    pl.kernel,
    out_type=jax.ShapeDtypeStruct((B,), jnp.float32),
    mesh=_mesh,
    scratch_types=[
        pltpu.VMEM((RBLK, C), jnp.float32),
        pltpu.VMEM((RBLK, C), jnp.float32),
        pltpu.VMEM((RPW,), jnp.int32),
        pltpu.VMEM((RPW,), jnp.float32),
        pltpu.SemaphoreType.DMA,
        pltpu.SemaphoreType.DMA,
    ],
    compiler_params=pltpu.CompilerParams(
        use_tc_tiling_on_sc=True, needs_layout_passes=False
    ),
)
def _cw_kernel(pred_hbm, y_hbm, out_hbm, buf_a, buf_b, y_v, loss_v, sem_a, sem_b):
    wid = lax.axis_index("s") * NC + lax.axis_index("c")
    base = pl.multiple_of(wid * RPW, RPW)
    pltpu.sync_copy(y_hbm.at[pl.ds(base, RPW)], y_v)

    neg_inf = jnp.full((LANES,), -jnp.inf, dtype=jnp.float32)
    lane = lax.iota(jnp.int32, LANES)

    def issue(g, buf, sem):
        row0 = pl.multiple_of(base + g * RBLK, RBLK)
        pltpu.async_copy(pred_hbm.at[pl.ds(row0, RBLK)], buf, sem)

    issue(0, buf_a, sem_a)
    issue(1, buf_b, sem_b)

    def top2_half(buf, row0):
        m1s = tuple(buf[row0 + r, pl.ds(C - LANES, LANES)] for r in range(HALF))
        m2s = (neg_inf,) * HALF

        def jbody(j, carry):
            m1s, m2s = carry
            c0 = pl.multiple_of(j * 32, 32)
            c1 = pl.multiple_of(j * 32 + 16, 16)
            for c in (c0, c1):
                vs = tuple(buf[row0 + r, pl.ds(c, LANES)] for r in range(HALF))
                m2s = tuple(
                    jnp.maximum(m2s[r], jnp.minimum(m1s[r], vs[r]))
                    for r in range(HALF)
                )
                m1s = tuple(jnp.maximum(m1s[r], vs[r]) for r in range(HALF))
            return m1s, m2s

        return lax.fori_loop(0, NPAIR, jbody, (m1s, m2s))

    def process(g, buf):
        for sub in range(RBLK // LANES):
            off = pl.multiple_of(g * RBLK + sub * LANES, LANES)
            yvec = y_v[pl.ds(off, LANES)]
            m1a, m2a = top2_half(buf, sub * LANES)
            m1b, m2b = top2_half(buf, sub * LANES + HALF)
            m1s = m1a + m1b
            m2s = m2a + m2b
            rm1 = neg_inf
            rm2 = neg_inf
            clsv = neg_inf
            for r in range(LANES):
                big1 = jnp.max(m1s[r])
                first = plsc.all_reduce_ffs(m1s[r] == big1)
                big2 = jnp.max(jnp.where(lane == first, m2s[r], m1s[r]))
                yr = yvec[r]
                cbase = pl.multiple_of((yr >> 4) << 4, LANES)
                chunk = buf[sub * LANES + r, pl.ds(cbase, LANES)]
                cls = jnp.max(jnp.where(lane == (yr & 15), chunk, neg_inf))
                rm1 = jnp.where(lane == r, big1, rm1)
                rm2 = jnp.where(lane == r, big2, rm2)
                clsv = jnp.where(lane == r, cls, clsv)
            target = jnp.where(clsv == rm1, rm2, rm1)
            loss_v[pl.ds(off, LANES)] = target - clsv

    def block_pair(h, carry):
        for par, buf, sem in ((0, buf_a, sem_a), (1, buf_b, sem_b)):
            g = 2 * h + par
            row0 = pl.multiple_of(base + g * RBLK, RBLK)
            pltpu.make_async_copy(pred_hbm.at[pl.ds(row0, RBLK)], buf, sem).wait()
            process(g, buf)

            @pl.when(g + 2 < NBLK)
            def _():
                issue(g + 2, buf, sem)

        return carry

    lax.fori_loop(0, NBLK // 2, block_pair, 0)
    pltpu.sync_copy(loss_v, out_hbm.at[pl.ds(base, RPW)])


def kernel(pred, y):
    return _cw_kernel(pred, y.astype(jnp.int32))
